# baseline (speedup 1.0000x reference)
import functools

import jax
import jax.numpy as jnp
from jax import lax
from jax.experimental import pallas as pl
from jax.experimental.pallas import tpu as pltpu
from jax.experimental.pallas import tpu_sc as plsc

N_EMB = 8192
DIM = 256
B = 8192
COMMIT = 0.25

BM = 2048
N_STEPS = B // BM
CK = 1024
N_CHUNKS = N_EMB // CK

NC = 2
NS = 16
NW = NC * NS
BPW = B // NW
CHUNK = 128


def _dist_argmin_kernel(z_ref, e_ref, idx_ref, loss_ref, en_ref, cols_ref):
    i = pl.program_id(0)

    @pl.when(i == 0)
    def _():
        en_ref[...] = jnp.sum(e_ref[...] * e_ref[...], axis=1)[None, :]
        cols_ref[...] = lax.broadcasted_iota(
            jnp.int32, (1, N_EMB), 1).astype(jnp.float32)
        loss_ref[0, 0] = 0.0

    z = z_ref[...]
    zn = jnp.sum(z * z, axis=1, keepdims=True)
    z2 = z + z
    best_m = None
    for c in range(N_CHUNKS):
        sl = pl.ds(c * CK, CK)
        mm2 = lax.dot_general(z2, e_ref[sl, :],
                              dimension_numbers=(((1,), (1,)), ((), ())),
                              preferred_element_type=jnp.float32)
        d = (zn + en_ref[:, sl]) - mm2
        m_c = jnp.min(d, axis=1, keepdims=True)
        i_c = jnp.min(jnp.where(d == m_c, cols_ref[:, sl],
                                jnp.float32(N_EMB)), axis=1, keepdims=True)
        if best_m is None:
            best_m, best_i = m_c, i_c
        else:
            best_i = jnp.where(m_c < best_m, i_c, best_i)
            best_m = jnp.minimum(best_m, m_c)
    idx_ref[...] = best_i[:, 0].astype(jnp.int32)[None, None, :]
    loss_ref[0, 0] += jnp.sum(best_m)

    @pl.when(i == pl.num_programs(0) - 1)
    def _():
        loss_ref[0, 0] = loss_ref[0, 0] * ((1.0 + COMMIT) / (B * DIM))


def _dist_argmin_call(inputs, emb, nb):
    nsteps = nb // BM
    return pl.pallas_call(
        _dist_argmin_kernel,
        grid=(nsteps,),
        in_specs=[
            pl.BlockSpec((BM, DIM), lambda i: (i, 0)),
            pl.BlockSpec((N_EMB, DIM), lambda i: (0, 0)),
        ],
        out_specs=[
            pl.BlockSpec((1, 1, BM), lambda i: (i, 0, 0)),
            pl.BlockSpec(memory_space=pltpu.SMEM, block_shape=(1, 1),
                         index_map=lambda i: (0, 0)),
        ],
        out_shape=[
            jax.ShapeDtypeStruct((nsteps, 1, BM), jnp.int32),
            jax.ShapeDtypeStruct((1, 1), jnp.float32),
        ],
        scratch_shapes=[pltpu.VMEM((1, N_EMB), jnp.float32),
                        pltpu.VMEM((1, N_EMB), jnp.float32)],
    )(inputs, emb)


def _make_sc_gather_body(nb):
    bpw = nb // NW
    nchunk = bpw // CHUNK

    def body(table_hbm, idx_hbm, out_hbm, idx_v, rows_v, sem):
        wid = lax.axis_index("s") * NC + lax.axis_index("c")
        pltpu.sync_copy(idx_hbm.at[pl.ds(wid * nchunk, nchunk)], idx_v)
        copies = [
            pltpu.async_copy(table_hbm.at[idx_v.at[j]],
                             rows_v.at[pl.ds(j * CHUNK, CHUNK)], sem)
            for j in range(nchunk)
        ]
        for c in copies:
            c.wait()
        pltpu.sync_copy(rows_v, out_hbm.at[pl.ds(wid * bpw, bpw)])

    return body


def _sc_gather(emb, idx2d, nb):
    call = pl.kernel(
        _make_sc_gather_body(nb),
        out_type=jax.ShapeDtypeStruct((nb, DIM), jnp.float32),
        mesh=plsc.VectorSubcoreMesh(core_axis_name="c", subcore_axis_name="s"),
        scratch_types=[
            pltpu.VMEM((nb // NW // CHUNK, CHUNK), jnp.int32),
            pltpu.VMEM((nb // NW, DIM), jnp.float32),
            pltpu.SemaphoreType.DMA,
        ],
    )
    return call(emb, idx2d)


def kernel(inputs, embedding_weight):
    h = B // 2
    parts = []
    for p in range(2):
        idx3, loss_p = _dist_argmin_call(
            inputs[p * h:(p + 1) * h], embedding_weight, h)
        idx_p = idx3.reshape(h)
        q_p = _sc_gather(embedding_weight,
                         idx_p.reshape(h // CHUNK, CHUNK), h)
        parts.append((idx_p, q_p, loss_p))
    idx = jnp.concatenate([parts[0][0], parts[1][0]])
    quantized = jnp.concatenate([parts[0][1], parts[1][1]])
    loss = (parts[0][2] + parts[1][2]).reshape(())
    return quantized, loss, idx


# speedup vs baseline: 1.3385x; 1.3385x over previous
import functools

import jax
import jax.numpy as jnp
from jax import lax
from jax.experimental import pallas as pl
from jax.experimental.pallas import tpu as pltpu
from jax.experimental.pallas import tpu_sc as plsc

N_EMB = 8192
DIM = 256
B = 8192
COMMIT = 0.25

BM = 2048
N_STEPS = B // BM
CK = 1024
N_CHUNKS = N_EMB // CK

NC = 2
NS = 16
NW = NC * NS
BPW = B // NW
CHUNK = 128


def _dist_argmin_kernel(z_ref, e_ref, idx_ref, loss_ref, en_ref, cols_ref):
    i = pl.program_id(0)

    @pl.when(i == 0)
    def _():
        en_ref[...] = jnp.sum(e_ref[...] * e_ref[...], axis=1)[None, :]
        cols_ref[...] = lax.broadcasted_iota(
            jnp.int32, (1, N_EMB), 1).astype(jnp.float32)
        loss_ref[0, 0] = 0.0

    z = z_ref[...]
    zn = jnp.sum(z * z, axis=1, keepdims=True)
    z2 = z + z
    best_m = None
    for c in range(N_CHUNKS):
        sl = pl.ds(c * CK, CK)
        mm2 = lax.dot_general(z2, e_ref[sl, :],
                              dimension_numbers=(((1,), (1,)), ((), ())),
                              preferred_element_type=jnp.float32)
        d = (zn + en_ref[:, sl]) - mm2
        m_c = jnp.min(d, axis=1, keepdims=True)
        i_c = jnp.min(jnp.where(d == m_c, cols_ref[:, sl],
                                jnp.float32(N_EMB)), axis=1, keepdims=True)
        if best_m is None:
            best_m, best_i = m_c, i_c
        else:
            best_i = jnp.where(m_c < best_m, i_c, best_i)
            best_m = jnp.minimum(best_m, m_c)
    idx_ref[...] = best_i[:, 0].astype(jnp.int32)[None, None, :]
    loss_ref[0, 0] += jnp.sum(best_m)

    @pl.when(i == pl.num_programs(0) - 1)
    def _():
        loss_ref[0, 0] = loss_ref[0, 0] * ((1.0 + COMMIT) / (B * DIM))


def _dist_argmin_call(inputs, emb, nb):
    nsteps = nb // BM
    return pl.pallas_call(
        _dist_argmin_kernel,
        grid=(nsteps,),
        in_specs=[
            pl.BlockSpec((BM, DIM), lambda i: (i, 0)),
            pl.BlockSpec((N_EMB, DIM), lambda i: (0, 0)),
        ],
        out_specs=[
            pl.BlockSpec((1, 1, BM), lambda i: (i, 0, 0)),
            pl.BlockSpec(memory_space=pltpu.SMEM, block_shape=(1, 1),
                         index_map=lambda i: (0, 0)),
        ],
        out_shape=[
            jax.ShapeDtypeStruct((nsteps, 1, BM), jnp.int32),
            jax.ShapeDtypeStruct((1, 1), jnp.float32),
        ],
        scratch_shapes=[pltpu.VMEM((1, N_EMB), jnp.float32),
                        pltpu.VMEM((1, N_EMB), jnp.float32)],
    )(inputs, emb)


def _make_sc_gather_body(nb):
    bpw = nb // NW
    nchunk = bpw // CHUNK

    def body(table_hbm, idx_hbm, out_hbm, idx_v, rows_v, sem):
        wid = lax.axis_index("s") * NC + lax.axis_index("c")
        pltpu.sync_copy(idx_hbm.at[pl.ds(wid * nchunk, nchunk)], idx_v)
        copies = [
            pltpu.async_copy(table_hbm.at[idx_v.at[j]],
                             rows_v.at[pl.ds(j * CHUNK, CHUNK)], sem)
            for j in range(nchunk)
        ]
        for c in copies:
            c.wait()
        pltpu.sync_copy(rows_v, out_hbm.at[pl.ds(wid * bpw, bpw)])

    return body


def _sc_gather(emb, idx2d, nb):
    call = pl.kernel(
        _make_sc_gather_body(nb),
        out_type=jax.ShapeDtypeStruct((nb, DIM), jnp.float32),
        mesh=plsc.VectorSubcoreMesh(core_axis_name="c", subcore_axis_name="s"),
        scratch_types=[
            pltpu.VMEM((nb // NW // CHUNK, CHUNK), jnp.int32),
            pltpu.VMEM((nb // NW, DIM), jnp.float32),
            pltpu.SemaphoreType.DMA,
        ],
    )
    return call(emb, idx2d)


def kernel(inputs, embedding_weight):
    idx3, loss = _dist_argmin_call(inputs, embedding_weight, B)
    idx = idx3.reshape(B)
    quantized = _sc_gather(embedding_weight,
                           idx.reshape(B // CHUNK, CHUNK), B)
    return quantized, loss.reshape(()), idx


# speedup vs baseline: 1.4211x; 1.0617x over previous
import functools

import jax
import jax.numpy as jnp
from jax import lax
from jax.experimental import pallas as pl
from jax.experimental.pallas import tpu as pltpu
from jax.experimental.pallas import tpu_sc as plsc

N_EMB = 8192
DIM = 256
B = 8192
COMMIT = 0.25

BM = 2048
N_STEPS = B // BM
CK = 1024
N_CHUNKS = N_EMB // CK

NC = 2
NS = 16
NW = NC * NS
BPW = B // NW
CHUNK = 128


def _dist_argmin_kernel(z_ref, e_ref, idx_ref, loss_ref, en_ref, cols_ref):
    i = pl.program_id(0)

    @pl.when(i == 0)
    def _():
        en_ref[...] = jnp.sum(e_ref[...] * e_ref[...], axis=1)[None, :]
        cols_ref[...] = lax.broadcasted_iota(
            jnp.int32, (1, N_EMB), 1).astype(jnp.float32)
        loss_ref[0, 0] = 0.0

    z = z_ref[...]
    zn = jnp.sum(z * z, axis=1, keepdims=True)
    z2 = z + z
    best_m = None
    for c in range(N_CHUNKS):
        sl = pl.ds(c * CK, CK)
        mm2 = lax.dot_general(z2, e_ref[sl, :],
                              dimension_numbers=(((1,), (1,)), ((), ())),
                              preferred_element_type=jnp.float32)
        d = (zn + en_ref[:, sl]) - mm2
        m_c = jnp.min(d, axis=1, keepdims=True)
        i_c = jnp.min(jnp.where(d == m_c, cols_ref[:, sl],
                                jnp.float32(N_EMB)), axis=1, keepdims=True)
        if best_m is None:
            best_m, best_i = m_c, i_c
        else:
            best_i = jnp.where(m_c < best_m, i_c, best_i)
            best_m = jnp.minimum(best_m, m_c)
    idx_ref[...] = best_i[:, 0].astype(jnp.int32).reshape(BM // CHUNK, CHUNK)
    loss_ref[0, 0] += jnp.sum(best_m)

    @pl.when(i == pl.num_programs(0) - 1)
    def _():
        loss_ref[0, 0] = loss_ref[0, 0] * ((1.0 + COMMIT) / (B * DIM))


def _dist_argmin_call(inputs, emb, nb):
    nsteps = nb // BM
    return pl.pallas_call(
        _dist_argmin_kernel,
        grid=(nsteps,),
        in_specs=[
            pl.BlockSpec((BM, DIM), lambda i: (i, 0)),
            pl.BlockSpec((N_EMB, DIM), lambda i: (0, 0)),
        ],
        out_specs=[
            pl.BlockSpec((BM // CHUNK, CHUNK), lambda i: (i, 0)),
            pl.BlockSpec(memory_space=pltpu.SMEM, block_shape=(1, 1),
                         index_map=lambda i: (0, 0)),
        ],
        out_shape=[
            jax.ShapeDtypeStruct((nb // CHUNK, CHUNK), jnp.int32),
            jax.ShapeDtypeStruct((1, 1), jnp.float32),
        ],
        scratch_shapes=[pltpu.VMEM((1, N_EMB), jnp.float32),
                        pltpu.VMEM((1, N_EMB), jnp.float32)],
    )(inputs, emb)


def _make_sc_gather_body(nb):
    bpw = nb // NW
    nchunk = bpw // CHUNK

    def body(table_hbm, idx_hbm, out_hbm, idx_v, rows_v, sem):
        wid = lax.axis_index("s") * NC + lax.axis_index("c")
        pltpu.sync_copy(idx_hbm.at[pl.ds(wid * nchunk, nchunk)], idx_v)
        copies = [
            pltpu.async_copy(table_hbm.at[idx_v.at[j]],
                             rows_v.at[pl.ds(j * CHUNK, CHUNK)], sem)
            for j in range(nchunk)
        ]
        for c in copies:
            c.wait()
        pltpu.sync_copy(rows_v, out_hbm.at[pl.ds(wid * bpw, bpw)])

    return body


def _sc_gather(emb, idx2d, nb):
    call = pl.kernel(
        _make_sc_gather_body(nb),
        out_type=jax.ShapeDtypeStruct((nb, DIM), jnp.float32),
        mesh=plsc.VectorSubcoreMesh(core_axis_name="c", subcore_axis_name="s"),
        scratch_types=[
            pltpu.VMEM((nb // NW // CHUNK, CHUNK), jnp.int32),
            pltpu.VMEM((nb // NW, DIM), jnp.float32),
            pltpu.SemaphoreType.DMA,
        ],
    )
    return call(emb, idx2d)


def kernel(inputs, embedding_weight):
    idx2d, loss = _dist_argmin_call(inputs, embedding_weight, B)
    quantized = _sc_gather(embedding_weight, idx2d, B)
    return quantized, loss.reshape(()), idx2d.reshape(B)
